# Initial kernel scaffold; baseline (speedup 1.0000x reference)
import jax
import jax.numpy as jnp
from jax.experimental import pallas as pl


def kernel(d_out_t, d_in_t, W_fwd, W_bwd):
    raise NotImplementedError("write your pallas kernel here")


# speedup vs baseline: 1.9047x; 1.9047x over previous
import functools

import jax
import jax.numpy as jnp
from jax.experimental import pallas as pl
from jax.experimental.pallas import tpu as pltpu

_K = 1024
_M = 32
_BLOCK_V = 1024


def _onehot_from_w(w):
    m, k = w.shape
    iota = jax.lax.broadcasted_iota(jnp.int32, (m, k), 1)
    mx = jnp.max(w, axis=1, keepdims=True)
    first = jnp.min(jnp.where(w == mx, iota, k), axis=1, keepdims=True)
    return (iota == first).astype(jnp.float32)


def _body(d_out_ref, d_in_ref, wf_ref, wb_ref, yf_ref, yb_ref):
    pf = _onehot_from_w(wf_ref[...])
    pb = _onehot_from_w(wb_ref[...])
    dn = (((1,), (1,)), ((), ()))
    yf_ref[...] = jax.lax.dot_general(
        d_out_ref[...], pf, dn, preferred_element_type=jnp.float32)
    yb_ref[...] = jax.lax.dot_general(
        d_in_ref[...], pb, dn, preferred_element_type=jnp.float32)


@jax.jit
def kernel(d_out_t, d_in_t, W_fwd, W_bwd):
    v = d_out_t.shape[0]
    grid = (pl.cdiv(v, _BLOCK_V),)
    d_spec = pl.BlockSpec((_BLOCK_V, _K), lambda i: (i, 0))
    w_spec = pl.BlockSpec((_M, _K), lambda i: (0, 0))
    y_spec = pl.BlockSpec((_BLOCK_V, _M), lambda i: (i, 0))
    yf, yb = pl.pallas_call(
        _body,
        grid=grid,
        in_specs=[d_spec, d_spec, w_spec, w_spec],
        out_specs=[y_spec, y_spec],
        out_shape=[
            jax.ShapeDtypeStruct((v, _M), jnp.float32),
            jax.ShapeDtypeStruct((v, _M), jnp.float32),
        ],
    )(d_out_t, d_in_t, W_fwd, W_bwd)
    return (yf, yb)
